# Initial kernel scaffold; baseline (speedup 1.0000x reference)
import jax
import jax.numpy as jnp
from jax.experimental import pallas as pl


def kernel(x, edge_index, W1, b1, W2, b2):
    raise NotImplementedError("write your pallas kernel here")


# speedup vs baseline: 3.4896x; 3.4896x over previous
import functools

import jax
import jax.numpy as jnp
from jax import lax
from jax.experimental import pallas as pl
from jax.experimental.pallas import tpu as pltpu
from jax.experimental.pallas import tpu_sc as plsc

N_NODES = 10000
N_EDGES = 320000
D = 128
NP = 10240
NC, NS = 2, 16
NW = NC * NS
L = 16
EDGE_BATCH = 128
BPW = 80
CHUNK = 16
NCHUNK = BPW // CHUNK
E_PAD = NW * BPW * EDGE_BATCH
ROWS_PER_TILE = NP // NS
BR = 1024
GRID = NP // BR


def _make_sc_pass(with_deg: bool):
    mesh = plsc.VectorSubcoreMesh(core_axis_name="c", subcore_axis_name="s")

    rows_t = jax.ShapeDtypeStruct((NC * NP, D), jnp.float32)
    deg_t = jax.ShapeDtypeStruct((NW, NP), jnp.float32)
    out_type = (rows_t, deg_t) if with_deg else rows_t
    scratch = [
        pltpu.VMEM((CHUNK, EDGE_BATCH), jnp.int32),
        pltpu.VMEM((CHUNK, EDGE_BATCH), jnp.int32),
        pltpu.VMEM((EDGE_BATCH, D), jnp.float32),
        pltpu.VMEM((EDGE_BATCH, D), jnp.float32),
        pltpu.VMEM_SHARED((NP, D), jnp.float32),
        pltpu.SemaphoreType.DMA,
        pltpu.SemaphoreType.DMA,
    ]
    if with_deg:
        scratch.append(pltpu.VMEM((NP,), jnp.float32))

    @functools.partial(
        pl.kernel, mesh=mesh, out_type=out_type, scratch_types=scratch,
        compiler_params=pltpu.CompilerParams(needs_layout_passes=False))
    def sc_pass(table_hbm, srcb_hbm, dstb_hbm, zeros_hbm, zdeg_hbm, *rest):
        if with_deg:
            out_hbm, deg_hbm, src_v, dst_v, msg0, msg1, agg_sh, sem0, sem1, \
                deg_v = rest
        else:
            out_hbm, src_v, dst_v, msg0, msg1, agg_sh, sem0, sem1 = rest

        c = lax.axis_index("c")
        s = lax.axis_index("s")
        wid = s * NC + c
        row0 = pl.multiple_of(s * ROWS_PER_TILE, ROWS_PER_TILE)

        pltpu.sync_copy(zeros_hbm.at[pl.ds(row0, ROWS_PER_TILE)],
                        agg_sh.at[pl.ds(row0, ROWS_PER_TILE)])
        if with_deg:
            pltpu.sync_copy(zdeg_hbm, deg_v)
        plsc.subcore_barrier()

        def chunk_body(g, _):
            bat0 = pl.multiple_of((wid * NCHUNK + g) * CHUNK, CHUNK)
            pltpu.sync_copy(srcb_hbm.at[pl.ds(bat0, CHUNK)], src_v)
            pltpu.sync_copy(dstb_hbm.at[pl.ds(bat0, CHUNK)], dst_v)

            pltpu.async_copy(table_hbm.at[src_v.at[0]], msg0, sem0)

            def body(b, _):
                @pl.when(b + 1 < CHUNK)
                def _():
                    @pl.when(lax.rem(b, 2) == 0)
                    def _():
                        pltpu.async_copy(table_hbm.at[src_v.at[b + 1]], msg1,
                                         sem1)

                    @pl.when(lax.rem(b, 2) == 1)
                    def _():
                        pltpu.async_copy(table_hbm.at[src_v.at[b + 1]], msg0,
                                         sem0)

                @pl.when(lax.rem(b, 2) == 0)
                def _():
                    pltpu.make_async_copy(table_hbm.at[src_v.at[0]], msg0,
                                          sem0).wait()
                    pltpu.sync_copy(msg0, agg_sh.at[dst_v.at[b]], add=True)

                @pl.when(lax.rem(b, 2) == 1)
                def _():
                    pltpu.make_async_copy(table_hbm.at[src_v.at[0]], msg1,
                                          sem1).wait()
                    pltpu.sync_copy(msg1, agg_sh.at[dst_v.at[b]], add=True)
                return 0

            lax.fori_loop(0, CHUNK, body, 0)

            if with_deg:
                ones16 = jnp.full((L,), 1.0, jnp.float32)

                def dbody(b, _):
                    for k in range(EDGE_BATCH // L):
                        idx16 = dst_v[b, pl.ds(k * L, L)]
                        plsc.addupdate_scatter(deg_v, [idx16], ones16)
                    return 0

                lax.fori_loop(0, CHUNK, dbody, 0)
            return 0

        lax.fori_loop(0, NCHUNK, chunk_body, 0)

        if with_deg:
            pltpu.sync_copy(deg_v, deg_hbm.at[wid])

        plsc.subcore_barrier()
        orow0 = pl.multiple_of(c * NP + s * ROWS_PER_TILE, ROWS_PER_TILE)
        pltpu.sync_copy(agg_sh.at[pl.ds(row0, ROWS_PER_TILE)],
                        out_hbm.at[pl.ds(orow0, ROWS_PER_TILE)])

    return sc_pass


_sc_pass_deg = _make_sc_pass(True)
_sc_pass = _make_sc_pass(False)


def _prep_body(x_ref, w_ref, o_ref):
    o_ref[...] = jnp.dot(x_ref[...], w_ref[...],
                         preferred_element_type=jnp.float32)


def _deg_col(dp_ref):
    return jnp.sum(dp_ref[...].T, axis=1, keepdims=True)


def _mid_body(p0_ref, p1_ref, t1_ref, dp_ref, b1_ref, w2_ref, o_ref):
    num = p0_ref[...] + p1_ref[...] + t1_ref[...]
    deg = _deg_col(dp_ref)
    h1 = jnp.maximum(num / (deg + 1.0) + b1_ref[...], 0.0)
    o_ref[...] = jnp.dot(h1, w2_ref[...], preferred_element_type=jnp.float32)


def _fin_body(q0_ref, q1_ref, t2_ref, dp_ref, b2_ref, o_ref):
    deg = _deg_col(dp_ref)
    o_ref[...] = (q0_ref[...] + q1_ref[...] + t2_ref[...]) / (deg + 1.0) \
        + b2_ref[...]


def kernel(x, edge_index, W1, b1, W2, b2):
    src = edge_index[0].astype(jnp.int32)
    dst = edge_index[1].astype(jnp.int32)
    n_pad_e = E_PAD - N_EDGES
    src_p = jnp.concatenate([src, jnp.zeros((n_pad_e,), jnp.int32)])
    dst_p = jnp.concatenate(
        [dst, N_NODES + (jnp.arange(n_pad_e, dtype=jnp.int32) % (NP - N_NODES))])
    srcb = src_p.reshape(NW * BPW, EDGE_BATCH)
    dstb = dst_p.reshape(NW * BPW, EDGE_BATCH)

    xp = jnp.pad(x, ((0, NP - N_NODES), (0, 0)))
    zeros128 = jnp.zeros((NP, D), jnp.float32)
    zdeg = jnp.zeros((NP,), jnp.float32)
    b1r = b1.reshape(1, D)
    b2r = b2.reshape(1, D)

    t1 = pl.pallas_call(
        _prep_body,
        grid=(GRID,),
        in_specs=[pl.BlockSpec((BR, D), lambda i: (i, 0)),
                  pl.BlockSpec((D, D), lambda i: (0, 0))],
        out_specs=pl.BlockSpec((BR, D), lambda i: (i, 0)),
        out_shape=jax.ShapeDtypeStruct((NP, D), jnp.float32),
    )(xp, W1)

    p, degp = _sc_pass_deg(t1, srcb, dstb, zeros128, zdeg)

    t2 = pl.pallas_call(
        _mid_body,
        grid=(GRID,),
        in_specs=[pl.BlockSpec((BR, D), lambda i: (i, 0)),
                  pl.BlockSpec((BR, D), lambda i: (i + GRID, 0)),
                  pl.BlockSpec((BR, D), lambda i: (i, 0)),
                  pl.BlockSpec((NW, BR), lambda i: (0, i)),
                  pl.BlockSpec((1, D), lambda i: (0, 0)),
                  pl.BlockSpec((D, D), lambda i: (0, 0))],
        out_specs=pl.BlockSpec((BR, D), lambda i: (i, 0)),
        out_shape=jax.ShapeDtypeStruct((NP, D), jnp.float32),
    )(p, p, t1, degp, b1r, W2)

    q = _sc_pass(t2, srcb, dstb, zeros128, zdeg)

    out = pl.pallas_call(
        _fin_body,
        grid=(GRID,),
        in_specs=[pl.BlockSpec((BR, D), lambda i: (i, 0)),
                  pl.BlockSpec((BR, D), lambda i: (i + GRID, 0)),
                  pl.BlockSpec((BR, D), lambda i: (i, 0)),
                  pl.BlockSpec((NW, BR), lambda i: (0, i)),
                  pl.BlockSpec((1, D), lambda i: (0, 0))],
        out_specs=pl.BlockSpec((BR, D), lambda i: (i, 0)),
        out_shape=jax.ShapeDtypeStruct((NP, D), jnp.float32),
    )(q, q, t2, degp, b2r)

    return out[:N_NODES]
